# baseline (speedup 1.0000x reference)
import functools

import jax
import jax.numpy as jnp
from jax import lax
from jax.experimental import pallas as pl
from jax.experimental.pallas import tpu as pltpu
from jax.experimental.pallas import tpu_sc as plsc

N_SPARSE = 26
N_DENSE = 13
N_OUT = N_SPARSE + N_DENSE
VOCAB = 100000
DIM = 32
B = 16384
LANES = 16

NC = 2
NS = 16
NW = NC * NS

TCOLS = VOCAB // 128
VTAIL = VOCAB - TCOLS * 128
NTCOL = N_SPARSE * TCOLS
K1_BLK = 2
K1_GRPS = NTCOL // K1_BLK
K1_ITERS = 321
LPF = VOCAB // 4
NLINES = N_SPARSE * LPF

TOTAL_ROWS = N_SPARSE * B
CHUNK = 128
PER_W = TOTAL_ROWS // NW
NCHUNK = PER_W // CHUNK
CPF = B // CHUNK
NBUF = 4
NGRP = NCHUNK // NBUF
DB = B // NW
DBH = DB // 2

_sc_mesh = plsc.VectorSubcoreMesh(core_axis_name="c", subcore_axis_name="s")
_params = pltpu.CompilerParams(use_tc_tiling_on_sc=True,
                               needs_layout_passes=False)


def _transpose_block(src_ref, dst_ref, nk, rows16):
    for k in range(nk):
        vals = []
        for r in range(4):
            for h in range(2):
                cols = jnp.broadcast_to(jnp.int32(4 * k + r), (LANES,))
                vals.append(plsc.load_gather(src_ref, [rows16[h], cols]))
        i = 0
        for r in range(4):
            for h in range(2):
                dst_ref[k, pl.ds(r * DIM + h * LANES, LANES)] = vals[i]
                i += 1


@functools.partial(
    pl.kernel,
    mesh=_sc_mesh,
    compiler_params=_params,
    out_type=jax.ShapeDtypeStruct((NLINES, 128), jnp.float32),
    scratch_types=[
        pltpu.VMEM((3, K1_BLK, DIM, 128), jnp.float32),
        pltpu.VMEM((3, K1_BLK, DIM, 128), jnp.float32),
        pltpu.VMEM((DIM, VTAIL), jnp.float32),
        pltpu.VMEM((VTAIL // 4, 128), jnp.float32),
        [pltpu.SemaphoreType.DMA] * 3,
        [pltpu.SemaphoreType.DMA] * 3,
        pltpu.SemaphoreType.DMA,
    ],
)
def _sc_repack(tab_hbm, lines_hbm, in_v, out_v, tin_v, tout_v,
               isems, osems, tsem):
    wid = lax.axis_index("s") * NC + lax.axis_index("c")
    i0 = jnp.arange(LANES, dtype=jnp.int32)
    rows16 = (i0, i0 + LANES)
    n_my = (K1_GRPS - wid + NW - 1) // NW

    @pl.when(wid < N_SPARSE)
    def _():
        f = wid
        pltpu.async_copy(
            tab_hbm.at[f, :, pl.ds(TCOLS * 128, VTAIL)], tin_v, tsem).wait()
        _transpose_block(tin_v, tout_v, VTAIL // 4, rows16)
        pltpu.async_copy(
            tout_v, lines_hbm.at[pl.ds(f * LPF + TCOLS * DIM, VTAIL // 4)],
            tsem).wait()

    def sub_copies(i, buf, ring_v, inward):
        res = []
        for s in range(K1_BLK):
            tc = (wid + i * NW) * K1_BLK + s
            f = tc // TCOLS
            c = tc % TCOLS
            hslice = tab_hbm.at[f, :, pl.ds(c * 128, 128)]
            lslice = lines_hbm.at[pl.ds(f * LPF + c * DIM, DIM)]
            if inward:
                res.append(pltpu.make_async_copy(
                    hslice, ring_v.at[buf, s], isems[buf]))
            else:
                res.append(pltpu.make_async_copy(
                    ring_v.at[buf, s], lslice, osems[buf]))
        return res

    for p in range(3):
        for d in sub_copies(p, p, in_v, True):
            d.start()

    def body(i, carry):
        for par in range(3):
            step = i * 3 + par

            @pl.when(step < n_my)
            def _():
                for d in sub_copies(step, par, in_v, True):
                    d.wait()

            @pl.when(jnp.logical_and(step >= 3, step - 3 < n_my))
            def _():
                for d in sub_copies(step - 3, par, out_v, False):
                    d.wait()

            for s in range(K1_BLK):
                _transpose_block(in_v.at[par, s], out_v.at[par, s],
                                 DIM, rows16)

            @pl.when(step < n_my)
            def _():
                for d in sub_copies(step, par, out_v, False):
                    d.start()

            @pl.when(step + 3 < n_my)
            def _():
                for d in sub_copies(step + 3, par, in_v, True):
                    d.start()
        return carry

    lax.fori_loop(0, K1_ITERS // 3, body, 0)


@functools.partial(
    pl.kernel,
    mesh=_sc_mesh,
    compiler_params=_params,
    out_type=jax.ShapeDtypeStruct((N_OUT, DIM, B), jnp.float32),
    scratch_types=[
        pltpu.VMEM((PER_W,), jnp.int32),
        pltpu.VMEM((NBUF, CHUNK), jnp.int32),
        pltpu.VMEM((NBUF, CHUNK, 128), jnp.float32),
        pltpu.VMEM((CHUNK,), jnp.int32),
        pltpu.VMEM((2, DIM, CHUNK), jnp.float32),
        pltpu.VMEM((N_DENSE * DB,), jnp.float32),
        pltpu.VMEM((N_DENSE * DIM,), jnp.float32),
        pltpu.VMEM((2, DIM, DBH), jnp.float32),
        [pltpu.SemaphoreType.DMA] * NBUF,
        [pltpu.SemaphoreType.DMA] * 2,
        pltpu.SemaphoreType.DMA,
        pltpu.SemaphoreType.DMA,
    ],
)
def _sc_lookup(lines_hbm, idx_hbm, x_hbm, w_hbm, out_hbm,
               idx_v, lidx_v, rows_v, sub_v, oblk_v, x_v, w_v, dblk_v,
               gsems, osems, dsem, ssem):
    wid = lax.axis_index("s") * NC + lax.axis_index("c")
    base = wid * PER_W
    dbase = wid * DB
    i0 = jnp.arange(LANES, dtype=jnp.int32)

    pltpu.sync_copy(idx_hbm.at[pl.ds(base, PER_W)], idx_v)
    for j in range(N_DENSE):
        pltpu.sync_copy(x_hbm.at[pl.ds(j * B + dbase, DB)],
                        x_v.at[pl.ds(j * DB, DB)])
    pltpu.sync_copy(w_hbm, w_v)

    def field_of(ci):
        return (wid * NCHUNK + ci) // CPF

    def b0_of(ci):
        return ((wid * NCHUNK + ci) % CPF) * CHUNK

    def start_gather(ci, b):
        f = field_of(ci)
        for g in range(CHUNK // LANES):
            v = idx_v[pl.ds(ci * CHUNK + g * LANES, LANES)]
            lidx_v[b, pl.ds(g * LANES, LANES)] = (
                f * LPF + jnp.right_shift(v, 2))
        pltpu.async_copy(lines_hbm.at[lidx_v.at[b]], rows_v.at[b], gsems[b])

    def emit_block(ci, b, ob):
        for g in range(CHUNK // LANES):
            v = idx_v[pl.ds(ci * CHUNK + g * LANES, LANES)]
            sub_v[pl.ds(g * LANES, LANES)] = (
                jnp.left_shift(jnp.bitwise_and(v, 3), 5))
        for g in range(CHUNK // LANES):
            rg = i0 + g * LANES
            sg = sub_v[pl.ds(g * LANES, LANES)]
            for d0 in range(0, DIM, 8):
                vals = [plsc.load_gather(rows_v.at[b], [rg, sg + (d0 + i)])
                        for i in range(8)]
                for i in range(8):
                    oblk_v[ob, d0 + i, pl.ds(g * LANES, LANES)] = vals[i]
        pltpu.async_copy(oblk_v.at[ob],
                         out_hbm.at[field_of(ci), :, pl.ds(b0_of(ci), CHUNK)],
                         osems[ob])

    for ci in range(NBUF):
        start_gather(ci, ci)

    def dense_body(t, carry):
        hb = t // DIM
        d = t % DIM
        j = hb // 2
        half = jnp.remainder(hb, 2)

        @pl.when(jnp.logical_and(d == 0, hb >= 2))
        def _():
            pltpu.make_async_copy(
                dblk_v.at[0], out_hbm.at[N_SPARSE, :, pl.ds(0, DBH)],
                dsem).wait()

        wjd = plsc.load_gather(
            w_v, [jnp.broadcast_to(j * DIM + d, (LANES,))])
        for g in range(DBH // LANES):
            xg = x_v[pl.ds(j * DB + half * DBH + g * LANES, LANES)]
            dblk_v[half, d, pl.ds(g * LANES, LANES)] = wjd * xg

        @pl.when(d == DIM - 1)
        def _():
            pltpu.async_copy(
                dblk_v.at[half],
                out_hbm.at[N_SPARSE + j, :,
                           pl.ds(dbase + half * DBH, DBH)], dsem)
        return carry

    lax.fori_loop(0, N_DENSE * 2 * DIM, dense_body, 0)

    def ring_body(go, carry):
        for bi in range(NBUF):
            ci = go * NBUF + bi
            ob = bi % 2
            pltpu.make_async_copy(
                lines_hbm.at[lidx_v.at[bi]], rows_v.at[bi], gsems[bi]).wait()

            @pl.when(ci >= 2)
            def _():
                pltpu.make_async_copy(
                    oblk_v.at[ob], out_hbm.at[0, :, pl.ds(0, CHUNK)],
                    osems[ob]).wait()

            emit_block(ci, bi, ob)

            @pl.when(ci + NBUF < NCHUNK)
            def _():
                start_gather(ci + NBUF, bi)
        return carry

    lax.fori_loop(0, NGRP, ring_body, 0)

    for last in (NCHUNK - 2, NCHUNK - 1):
        pltpu.make_async_copy(
            oblk_v.at[last % 2], out_hbm.at[0, :, pl.ds(0, CHUNK)],
            osems[last % 2]).wait()
    for _ in range(2):
        pltpu.make_async_copy(
            dblk_v.at[0], out_hbm.at[N_SPARSE, :, pl.ds(0, DBH)], dsem).wait()


def kernel(sparse_inputs, dense_inputs, sparse_weights, dense_weights):
    tab_t = jnp.transpose(sparse_weights, (0, 2, 1))
    idx_flat = sparse_inputs[:, :, 0].astype(jnp.int32).reshape(TOTAL_ROWS)
    x_flat = dense_inputs[:, :, 0].reshape(N_DENSE * B)
    w_flat = dense_weights.reshape(N_DENSE * DIM)

    lines = _sc_repack(tab_t)
    out_t = _sc_lookup(lines, idx_flat, x_flat, w_flat)
    return jnp.transpose(out_t, (0, 2, 1))


# speedup vs baseline: 1.3419x; 1.3419x over previous
import functools

import jax
import jax.numpy as jnp
from jax import lax
from jax.experimental import pallas as pl
from jax.experimental.pallas import tpu as pltpu
from jax.experimental.pallas import tpu_sc as plsc

N_SPARSE = 26
N_DENSE = 13
N_OUT = N_SPARSE + N_DENSE
VOCAB = 100000
DIM = 32
B = 16384
LANES = 16

NC = 2
NS = 16
NW = NC * NS

CVOC = 2048
NCHK = 49
LPF = NCHK * CVOC // 4
NLINES = N_SPARSE * LPF

TOTAL_ROWS = N_SPARSE * B
CHUNK = 128
PER_W = TOTAL_ROWS // NW
NCHUNK = PER_W // CHUNK
CPF = B // CHUNK
NBUF = 4
NGRP = NCHUNK // NBUF
DB = B // NW
DBH = DB // 2

_sc_mesh = plsc.VectorSubcoreMesh(core_axis_name="c", subcore_axis_name="s")
_params = pltpu.CompilerParams(use_tc_tiling_on_sc=True,
                               needs_layout_passes=False)


QV = CVOC // 4


def _tc_repack_body(tab_ref, out_ref):
    at = tab_ref[0].T
    out_ref[...] = jnp.concatenate(
        [at[q * QV:(q + 1) * QV, :] for q in range(4)], axis=1)


_tc_repack = pl.pallas_call(
    _tc_repack_body,
    grid=(N_SPARSE, NCHK),
    in_specs=[pl.BlockSpec((1, DIM, CVOC), lambda f, c: (f, 0, c))],
    out_specs=pl.BlockSpec((CVOC // 4, 128), lambda f, c: (f * NCHK + c, 0)),
    out_shape=jax.ShapeDtypeStruct((NLINES, 128), jnp.float32),
)


@functools.partial(
    pl.kernel,
    mesh=_sc_mesh,
    compiler_params=_params,
    out_type=jax.ShapeDtypeStruct((N_OUT, DIM, B), jnp.float32),
    scratch_types=[
        pltpu.VMEM((PER_W,), jnp.int32),
        pltpu.VMEM((NBUF, CHUNK), jnp.int32),
        pltpu.VMEM((NBUF, CHUNK, 128), jnp.float32),
        pltpu.VMEM((CHUNK,), jnp.int32),
        pltpu.VMEM((2, DIM, CHUNK), jnp.float32),
        pltpu.VMEM((N_DENSE * DB,), jnp.float32),
        pltpu.VMEM((N_DENSE * DIM,), jnp.float32),
        pltpu.VMEM((2, DIM, DBH), jnp.float32),
        [pltpu.SemaphoreType.DMA] * NBUF,
        [pltpu.SemaphoreType.DMA] * 2,
        pltpu.SemaphoreType.DMA,
        pltpu.SemaphoreType.DMA,
    ],
)
def _sc_lookup(lines_hbm, idx_hbm, x_hbm, w_hbm, out_hbm,
               idx_v, lidx_v, rows_v, sub_v, oblk_v, x_v, w_v, dblk_v,
               gsems, osems, dsem, ssem):
    wid = lax.axis_index("s") * NC + lax.axis_index("c")
    base = wid * PER_W
    dbase = wid * DB
    i0 = jnp.arange(LANES, dtype=jnp.int32)

    pltpu.sync_copy(idx_hbm.at[pl.ds(base, PER_W)], idx_v)
    for j in range(N_DENSE):
        pltpu.sync_copy(x_hbm.at[pl.ds(j * B + dbase, DB)],
                        x_v.at[pl.ds(j * DB, DB)])
    pltpu.sync_copy(w_hbm, w_v)

    def field_of(ci):
        return (wid * NCHUNK + ci) // CPF

    def b0_of(ci):
        return ((wid * NCHUNK + ci) % CPF) * CHUNK

    def start_gather(ci, b):
        f = field_of(ci)
        for g in range(CHUNK // LANES):
            v = idx_v[pl.ds(ci * CHUNK + g * LANES, LANES)]
            lidx_v[b, pl.ds(g * LANES, LANES)] = (
                f * LPF + jnp.left_shift(jnp.right_shift(v, 11), 9)
                + jnp.bitwise_and(v, 511))
        pltpu.async_copy(lines_hbm.at[lidx_v.at[b]], rows_v.at[b], gsems[b])

    def emit_block(ci, b, ob):
        for g in range(CHUNK // LANES):
            v = idx_v[pl.ds(ci * CHUNK + g * LANES, LANES)]
            sub_v[pl.ds(g * LANES, LANES)] = jnp.left_shift(
                jnp.bitwise_and(jnp.right_shift(v, 9), 3), 5)
        for g in range(CHUNK // LANES):
            rg = i0 + g * LANES
            sg = sub_v[pl.ds(g * LANES, LANES)]
            for d0 in range(0, DIM, 8):
                vals = [plsc.load_gather(rows_v.at[b], [rg, sg + (d0 + i)])
                        for i in range(8)]
                for i in range(8):
                    oblk_v[ob, d0 + i, pl.ds(g * LANES, LANES)] = vals[i]
        pltpu.async_copy(oblk_v.at[ob],
                         out_hbm.at[field_of(ci), :, pl.ds(b0_of(ci), CHUNK)],
                         osems[ob])

    for ci in range(NBUF):
        start_gather(ci, ci)

    def dense_body(t, carry):
        hb = t // DIM
        d = t % DIM
        j = hb // 2
        half = jnp.remainder(hb, 2)

        @pl.when(jnp.logical_and(d == 0, hb >= 2))
        def _():
            pltpu.make_async_copy(
                dblk_v.at[0], out_hbm.at[N_SPARSE, :, pl.ds(0, DBH)],
                dsem).wait()

        wjd = plsc.load_gather(
            w_v, [jnp.broadcast_to(j * DIM + d, (LANES,))])
        for g in range(DBH // LANES):
            xg = x_v[pl.ds(j * DB + half * DBH + g * LANES, LANES)]
            dblk_v[half, d, pl.ds(g * LANES, LANES)] = wjd * xg

        @pl.when(d == DIM - 1)
        def _():
            pltpu.async_copy(
                dblk_v.at[half],
                out_hbm.at[N_SPARSE + j, :,
                           pl.ds(dbase + half * DBH, DBH)], dsem)
        return carry

    lax.fori_loop(0, N_DENSE * 2 * DIM, dense_body, 0)

    def ring_body(go, carry):
        for bi in range(NBUF):
            ci = go * NBUF + bi
            ob = bi % 2
            pltpu.make_async_copy(
                lines_hbm.at[lidx_v.at[bi]], rows_v.at[bi], gsems[bi]).wait()

            @pl.when(ci >= 2)
            def _():
                pltpu.make_async_copy(
                    oblk_v.at[ob], out_hbm.at[0, :, pl.ds(0, CHUNK)],
                    osems[ob]).wait()

            emit_block(ci, bi, ob)

            @pl.when(ci + NBUF < NCHUNK)
            def _():
                start_gather(ci + NBUF, bi)
        return carry

    lax.fori_loop(0, NGRP, ring_body, 0)

    for last in (NCHUNK - 2, NCHUNK - 1):
        pltpu.make_async_copy(
            oblk_v.at[last % 2], out_hbm.at[0, :, pl.ds(0, CHUNK)],
            osems[last % 2]).wait()
    for _ in range(2):
        pltpu.make_async_copy(
            dblk_v.at[0], out_hbm.at[N_SPARSE, :, pl.ds(0, DBH)], dsem).wait()


def kernel(sparse_inputs, dense_inputs, sparse_weights, dense_weights):
    tab_t = jnp.transpose(sparse_weights, (0, 2, 1))
    idx_flat = sparse_inputs[:, :, 0].astype(jnp.int32).reshape(TOTAL_ROWS)
    x_flat = dense_inputs[:, :, 0].reshape(N_DENSE * B)
    w_flat = dense_weights.reshape(N_DENSE * DIM)

    lines = _tc_repack(tab_t)
    out_t = _sc_lookup(lines, idx_flat, x_flat, w_flat)
    return jnp.transpose(out_t, (0, 2, 1))


# speedup vs baseline: 1.8735x; 1.3962x over previous
import functools

import jax
import jax.numpy as jnp
from jax import lax
from jax.experimental import pallas as pl
from jax.experimental.pallas import tpu as pltpu
from jax.experimental.pallas import tpu_sc as plsc

N_SPARSE = 26
N_DENSE = 13
N_OUT = N_SPARSE + N_DENSE
VOCAB = 100000
DIM = 32
B = 16384
LANES = 16

NC = 2
NS = 16
NW = NC * NS

SUBW = 2048
SUB = 4
CVOC = SUB * SUBW
NCHK = 13
LPF = NCHK * CVOC // 4
NLINES = N_SPARSE * LPF

TOTAL_ROWS = N_SPARSE * B
CHUNK = 128
PER_W = TOTAL_ROWS // NW
NCHUNK = PER_W // CHUNK
CPF = B // CHUNK
NBUF = 4
NGRP = NCHUNK // NBUF
DB = B // NW
DBH = DB // 2

_sc_mesh = plsc.VectorSubcoreMesh(core_axis_name="c", subcore_axis_name="s")
_params = pltpu.CompilerParams(use_tc_tiling_on_sc=True,
                               needs_layout_passes=False)


QV = SUBW // 4


def _tc_repack_body(tab_ref, out_ref):
    a = tab_ref[0]
    for s in range(SUB):
        at = a[:, s * SUBW:(s + 1) * SUBW].T
        out_ref[s * QV:(s + 1) * QV, :] = jnp.concatenate(
            [at[q * QV:(q + 1) * QV, :] for q in range(4)], axis=1)


_tc_repack = pl.pallas_call(
    _tc_repack_body,
    grid=(N_SPARSE, NCHK),
    in_specs=[pl.BlockSpec((1, DIM, CVOC), lambda f, c: (f, 0, c))],
    out_specs=pl.BlockSpec((CVOC // 4, 128), lambda f, c: (f * NCHK + c, 0)),
    out_shape=jax.ShapeDtypeStruct((NLINES, 128), jnp.float32),
)


@functools.partial(
    pl.kernel,
    mesh=_sc_mesh,
    compiler_params=_params,
    out_type=jax.ShapeDtypeStruct((N_OUT, DIM, B), jnp.float32),
    scratch_types=[
        pltpu.VMEM((PER_W,), jnp.int32),
        pltpu.VMEM((NBUF, CHUNK), jnp.int32),
        pltpu.VMEM((NBUF, CHUNK, 128), jnp.float32),
        pltpu.VMEM((CHUNK,), jnp.int32),
        pltpu.VMEM((2, DIM, CHUNK), jnp.float32),
        pltpu.VMEM((N_DENSE * DB,), jnp.float32),
        pltpu.VMEM((N_DENSE * DIM,), jnp.float32),
        pltpu.VMEM((2, DIM, DBH), jnp.float32),
        [pltpu.SemaphoreType.DMA] * NBUF,
        [pltpu.SemaphoreType.DMA] * 2,
        pltpu.SemaphoreType.DMA,
        pltpu.SemaphoreType.DMA,
    ],
)
def _sc_lookup(lines_hbm, idx_hbm, x_hbm, w_hbm, out_hbm,
               idx_v, lidx_v, rows_v, sub_v, oblk_v, x_v, w_v, dblk_v,
               gsems, osems, dsem, ssem):
    wid = lax.axis_index("s") * NC + lax.axis_index("c")
    base = wid * PER_W
    dbase = wid * DB
    i0 = jnp.arange(LANES, dtype=jnp.int32)

    pltpu.sync_copy(idx_hbm.at[pl.ds(base, PER_W)], idx_v)
    for j in range(N_DENSE):
        pltpu.sync_copy(x_hbm.at[pl.ds(j * B + dbase, DB)],
                        x_v.at[pl.ds(j * DB, DB)])
    pltpu.sync_copy(w_hbm, w_v)

    def field_of(ci):
        return (wid * NCHUNK + ci) // CPF

    def b0_of(ci):
        return ((wid * NCHUNK + ci) % CPF) * CHUNK

    def start_gather(ci, b):
        f = field_of(ci)
        for g in range(CHUNK // LANES):
            v = idx_v[pl.ds(ci * CHUNK + g * LANES, LANES)]
            lidx_v[b, pl.ds(g * LANES, LANES)] = (
                f * LPF + jnp.left_shift(jnp.right_shift(v, 11), 9)
                + jnp.bitwise_and(v, 511))
        pltpu.async_copy(lines_hbm.at[lidx_v.at[b]], rows_v.at[b], gsems[b])

    def emit_block(ci, b, ob):
        for g in range(CHUNK // LANES):
            v = idx_v[pl.ds(ci * CHUNK + g * LANES, LANES)]
            sub_v[pl.ds(g * LANES, LANES)] = jnp.left_shift(
                jnp.bitwise_and(jnp.right_shift(v, 9), 3), 5)
        for g in range(CHUNK // LANES):
            rg = i0 + g * LANES
            sg = sub_v[pl.ds(g * LANES, LANES)]
            for d0 in range(0, DIM, 8):
                vals = [plsc.load_gather(rows_v.at[b], [rg, sg + (d0 + i)])
                        for i in range(8)]
                for i in range(8):
                    oblk_v[ob, d0 + i, pl.ds(g * LANES, LANES)] = vals[i]
        pltpu.async_copy(oblk_v.at[ob],
                         out_hbm.at[field_of(ci), :, pl.ds(b0_of(ci), CHUNK)],
                         osems[ob])

    for ci in range(NBUF):
        start_gather(ci, ci)

    def dense_body(t, carry):
        hb = t // DIM
        d = t % DIM
        j = hb // 2
        half = jnp.remainder(hb, 2)

        @pl.when(jnp.logical_and(d == 0, hb >= 2))
        def _():
            pltpu.make_async_copy(
                dblk_v.at[0], out_hbm.at[N_SPARSE, :, pl.ds(0, DBH)],
                dsem).wait()

        wjd = plsc.load_gather(
            w_v, [jnp.broadcast_to(j * DIM + d, (LANES,))])
        for g in range(DBH // LANES):
            xg = x_v[pl.ds(j * DB + half * DBH + g * LANES, LANES)]
            dblk_v[half, d, pl.ds(g * LANES, LANES)] = wjd * xg

        @pl.when(d == DIM - 1)
        def _():
            pltpu.async_copy(
                dblk_v.at[half],
                out_hbm.at[N_SPARSE + j, :,
                           pl.ds(dbase + half * DBH, DBH)], dsem)
        return carry

    lax.fori_loop(0, N_DENSE * 2 * DIM, dense_body, 0)

    def ring_body(go, carry):
        for bi in range(NBUF):
            ci = go * NBUF + bi
            ob = bi % 2
            pltpu.make_async_copy(
                lines_hbm.at[lidx_v.at[bi]], rows_v.at[bi], gsems[bi]).wait()

            @pl.when(ci >= 2)
            def _():
                pltpu.make_async_copy(
                    oblk_v.at[ob], out_hbm.at[0, :, pl.ds(0, CHUNK)],
                    osems[ob]).wait()

            emit_block(ci, bi, ob)

            @pl.when(ci + NBUF < NCHUNK)
            def _():
                start_gather(ci + NBUF, bi)
        return carry

    lax.fori_loop(0, NGRP, ring_body, 0)

    for last in (NCHUNK - 2, NCHUNK - 1):
        pltpu.make_async_copy(
            oblk_v.at[last % 2], out_hbm.at[0, :, pl.ds(0, CHUNK)],
            osems[last % 2]).wait()
    for _ in range(2):
        pltpu.make_async_copy(
            dblk_v.at[0], out_hbm.at[N_SPARSE, :, pl.ds(0, DBH)], dsem).wait()


def kernel(sparse_inputs, dense_inputs, sparse_weights, dense_weights):
    tab_t = jnp.transpose(sparse_weights, (0, 2, 1))
    idx_flat = sparse_inputs[:, :, 0].astype(jnp.int32).reshape(TOTAL_ROWS)
    x_flat = dense_inputs[:, :, 0].reshape(N_DENSE * B)
    w_flat = dense_weights.reshape(N_DENSE * DIM)

    lines = _tc_repack(tab_t)
    out_t = _sc_lookup(lines, idx_flat, x_flat, w_flat)
    return jnp.transpose(out_t, (0, 2, 1))


# speedup vs baseline: 2.7263x; 1.4552x over previous
import functools

import jax
import jax.numpy as jnp
from jax import lax
from jax.experimental import pallas as pl
from jax.experimental.pallas import tpu as pltpu
from jax.experimental.pallas import tpu_sc as plsc

N_SPARSE = 26
N_DENSE = 13
N_OUT = N_SPARSE + N_DENSE
VOCAB = 100000
DIM = 32
B = 16384
LANES = 16

NC = 2
NS = 16
NW = NC * NS

SUBW = 2048
SUB = 4
CVOC = SUB * SUBW
NCHK = 13
LPF = NCHK * CVOC // 4
NLINES = N_SPARSE * LPF

TOTAL_ROWS = N_SPARSE * B
CHUNK = 128
PER_W = TOTAL_ROWS // NW
NCHUNK = PER_W // CHUNK
CPF = B // CHUNK
NBUF = 4
NGRP = NCHUNK // NBUF
DB = B // NW
DBH = DB // 2

_sc_mesh = plsc.VectorSubcoreMesh(core_axis_name="c", subcore_axis_name="s")
_params = pltpu.CompilerParams(use_tc_tiling_on_sc=True,
                               needs_layout_passes=False)


QV = SUBW // 4


def _tc_repack_body(tab_ref, out_ref):
    a = tab_ref[0]
    for s in range(SUB):
        b = jnp.concatenate(
            [a[:, s * SUBW + q * QV:s * SUBW + (q + 1) * QV]
             for q in range(4)], axis=0)
        out_ref[s * QV:(s + 1) * QV, :] = b.T


_tc_repack = pl.pallas_call(
    _tc_repack_body,
    grid=(N_SPARSE, NCHK),
    in_specs=[pl.BlockSpec((1, DIM, CVOC), lambda f, c: (f, 0, c))],
    out_specs=pl.BlockSpec((CVOC // 4, 128), lambda f, c: (f * NCHK + c, 0)),
    out_shape=jax.ShapeDtypeStruct((NLINES, 128), jnp.float32),
)


@functools.partial(
    pl.kernel,
    mesh=_sc_mesh,
    compiler_params=_params,
    out_type=jax.ShapeDtypeStruct((N_OUT, DIM, B), jnp.float32),
    scratch_types=[
        pltpu.VMEM((PER_W,), jnp.int32),
        pltpu.VMEM((NBUF, CHUNK), jnp.int32),
        pltpu.VMEM((NBUF, CHUNK, 128), jnp.float32),
        pltpu.VMEM((CHUNK,), jnp.int32),
        pltpu.VMEM((2, DIM, CHUNK), jnp.float32),
        pltpu.VMEM((N_DENSE * DB,), jnp.float32),
        pltpu.VMEM((N_DENSE * DIM,), jnp.float32),
        pltpu.VMEM((2, DIM, DBH), jnp.float32),
        [pltpu.SemaphoreType.DMA] * NBUF,
        [pltpu.SemaphoreType.DMA] * 2,
        pltpu.SemaphoreType.DMA,
        pltpu.SemaphoreType.DMA,
    ],
)
def _sc_lookup(lines_hbm, idx_hbm, x_hbm, w_hbm, out_hbm,
               idx_v, lidx_v, rows_v, sub_v, oblk_v, x_v, w_v, dblk_v,
               gsems, osems, dsem, ssem):
    wid = lax.axis_index("s") * NC + lax.axis_index("c")
    base = wid * PER_W
    dbase = wid * DB
    i0 = jnp.arange(LANES, dtype=jnp.int32)

    pltpu.sync_copy(idx_hbm.at[pl.ds(base, PER_W)], idx_v)
    for j in range(N_DENSE):
        pltpu.sync_copy(x_hbm.at[pl.ds(j * B + dbase, DB)],
                        x_v.at[pl.ds(j * DB, DB)])
    pltpu.sync_copy(w_hbm, w_v)

    def field_of(ci):
        return (wid * NCHUNK + ci) // CPF

    def b0_of(ci):
        return ((wid * NCHUNK + ci) % CPF) * CHUNK

    def start_gather(ci, b):
        f = field_of(ci)
        for g in range(CHUNK // LANES):
            v = idx_v[pl.ds(ci * CHUNK + g * LANES, LANES)]
            lidx_v[b, pl.ds(g * LANES, LANES)] = (
                f * LPF + jnp.left_shift(jnp.right_shift(v, 11), 9)
                + jnp.bitwise_and(v, 511))
        pltpu.async_copy(lines_hbm.at[lidx_v.at[b]], rows_v.at[b], gsems[b])

    def emit_block(ci, b, ob):
        for g in range(CHUNK // LANES):
            v = idx_v[pl.ds(ci * CHUNK + g * LANES, LANES)]
            sub_v[pl.ds(g * LANES, LANES)] = jnp.left_shift(
                jnp.bitwise_and(jnp.right_shift(v, 9), 3), 5)
        for g in range(CHUNK // LANES):
            rg = i0 + g * LANES
            sg = sub_v[pl.ds(g * LANES, LANES)]
            for d0 in range(0, DIM, 8):
                vals = [plsc.load_gather(rows_v.at[b], [rg, sg + (d0 + i)])
                        for i in range(8)]
                for i in range(8):
                    oblk_v[ob, d0 + i, pl.ds(g * LANES, LANES)] = vals[i]
        pltpu.async_copy(oblk_v.at[ob],
                         out_hbm.at[field_of(ci), :, pl.ds(b0_of(ci), CHUNK)],
                         osems[ob])

    for ci in range(NBUF):
        start_gather(ci, ci)

    def dense_body(t, carry):
        hb = t // DIM
        d = t % DIM
        j = hb // 2
        half = jnp.remainder(hb, 2)

        @pl.when(jnp.logical_and(d == 0, hb >= 2))
        def _():
            pltpu.make_async_copy(
                dblk_v.at[0], out_hbm.at[N_SPARSE, :, pl.ds(0, DBH)],
                dsem).wait()

        wjd = plsc.load_gather(
            w_v, [jnp.broadcast_to(j * DIM + d, (LANES,))])
        for g in range(DBH // LANES):
            xg = x_v[pl.ds(j * DB + half * DBH + g * LANES, LANES)]
            dblk_v[half, d, pl.ds(g * LANES, LANES)] = wjd * xg

        @pl.when(d == DIM - 1)
        def _():
            pltpu.async_copy(
                dblk_v.at[half],
                out_hbm.at[N_SPARSE + j, :,
                           pl.ds(dbase + half * DBH, DBH)], dsem)
        return carry

    lax.fori_loop(0, N_DENSE * 2 * DIM, dense_body, 0)

    def ring_body(go, carry):
        for bi in range(NBUF):
            ci = go * NBUF + bi
            ob = bi % 2
            pltpu.make_async_copy(
                lines_hbm.at[lidx_v.at[bi]], rows_v.at[bi], gsems[bi]).wait()

            @pl.when(ci >= 2)
            def _():
                pltpu.make_async_copy(
                    oblk_v.at[ob], out_hbm.at[0, :, pl.ds(0, CHUNK)],
                    osems[ob]).wait()

            emit_block(ci, bi, ob)

            @pl.when(ci + NBUF < NCHUNK)
            def _():
                start_gather(ci + NBUF, bi)
        return carry

    lax.fori_loop(0, NGRP, ring_body, 0)

    for last in (NCHUNK - 2, NCHUNK - 1):
        pltpu.make_async_copy(
            oblk_v.at[last % 2], out_hbm.at[0, :, pl.ds(0, CHUNK)],
            osems[last % 2]).wait()
    for _ in range(2):
        pltpu.make_async_copy(
            dblk_v.at[0], out_hbm.at[N_SPARSE, :, pl.ds(0, DBH)], dsem).wait()


def kernel(sparse_inputs, dense_inputs, sparse_weights, dense_weights):
    tab_t = jnp.transpose(sparse_weights, (0, 2, 1))
    idx_flat = sparse_inputs[:, :, 0].astype(jnp.int32).reshape(TOTAL_ROWS)
    x_flat = dense_inputs[:, :, 0].reshape(N_DENSE * B)
    w_flat = dense_weights.reshape(N_DENSE * DIM)

    lines = _tc_repack(tab_t)
    out_t = _sc_lookup(lines, idx_flat, x_flat, w_flat)
    return jnp.transpose(out_t, (0, 2, 1))
